# baseline (speedup 1.0000x reference)
import functools

import jax
import jax.numpy as jnp
from jax import lax
from jax.experimental import pallas as pl
from jax.experimental.pallas import tpu as pltpu
from jax.experimental.pallas import tpu_sc as plsc

_NC = 2
_NS = 16
_NW = _NC * _NS
_L = 16
_C = 48
_NB = 3
_D = 256
_DW = _D // 2


@functools.lru_cache(maxsize=None)
def _make_kernel(E, N):
    epw = E // _NW
    nt = epw // _C
    tail = epw - nt * _C
    assert E % _NW == 0 and tail % 8 == 0 and 0 < tail <= _L
    mesh = plsc.VectorSubcoreMesh(core_axis_name="c", subcore_axis_name="s")

    @functools.partial(
        pl.kernel,
        out_type=jax.ShapeDtypeStruct((E,), jnp.float32),
        mesh=mesh,
        compiler_params=pltpu.CompilerParams(
            needs_layout_passes=False,
            internal_scratch_in_bytes=1024 * 1024),
        scratch_types=[
            pltpu.VMEM((epw,), jnp.int32),
            pltpu.VMEM((epw,), jnp.int32),
            pltpu.VMEM((_NB, _C, _DW), jnp.int32),
            pltpu.VMEM((_NB, _C, _DW), jnp.int32),
            pltpu.VMEM((epw + _L - tail,), jnp.float32),
            pltpu.VMEM((_L * _L,), jnp.float32),
        ] + [pltpu.SemaphoreType.DMA] * _NB,
    )
    def ker(emb, src, dst, out, sidx, didx, srows, drows, scores, accbuf,
            *sems):
        sid = lax.axis_index("s")
        wid = sid * _NC + lax.axis_index("c")
        base = wid * epw

        pltpu.sync_copy(src.at[pl.ds(base, epw)], sidx)
        pltpu.sync_copy(dst.at[pl.ds(base, epw)], didx)

        def fire(g, b, n):
            pltpu.async_copy(emb.at[sidx.at[pl.ds(g * _C, n)]],
                             srows.at[b, pl.ds(0, n)], sems[b])
            pltpu.async_copy(emb.at[didx.at[pl.ds(g * _C, n)]],
                             drows.at[b, pl.ds(0, n)], sems[b])

        def drain(g, b, n):
            pltpu.make_async_copy(emb.at[sidx.at[pl.ds(g * _C, n)]],
                                  srows.at[b, pl.ds(0, n)], sems[b]).wait()
            pltpu.make_async_copy(emb.at[didx.at[pl.ds(g * _C, n)]],
                                  drows.at[b, pl.ds(0, n)], sems[b]).wait()

        def dot_group(b, j, lanes):
            for m in range(lanes):
                e = j * _L + m
                acc = None
                for k in range(_DW // _L):
                    sv = plsc.bitcast(srows[b, e, pl.ds(k * _L, _L)],
                                      jnp.bfloat16)
                    dv = plsc.bitcast(drows[b, e, pl.ds(k * _L, _L)],
                                      jnp.bfloat16)
                    u, v = plsc.unpack(sv * dv,
                                       format=plsc.PackFormat.INTERLEAVED)
                    acc = u + v if acc is None else acc + u + v
                accbuf[pl.ds(m * _L, _L)] = acc
            iot = lax.iota(jnp.int32, _L) * _L
            svec = plsc.load_gather(accbuf, [iot])
            for l in range(1, _L):
                svec = svec + plsc.load_gather(accbuf, [iot + l])
            return svec

        for p in range(_NB - 1):
            if p < nt:
                fire(p, p, _C)

        @pl.loop(0, nt, step=_NB)
        def _chunks(t):
            for b in range(_NB):
                g = t + b

                @pl.when(g < nt)
                def _():
                    @pl.when(g + _NB - 1 < nt)
                    def _():
                        fire(g + _NB - 1, (b + _NB - 1) % _NB, _C)

                    drain(g, b, _C)

                    @pl.loop(0, _C // _L)
                    def _groups(j):
                        scores[pl.ds(g * _C + j * _L, _L)] = \
                            dot_group(b, j, _L)

        fire(nt, 0, tail)
        drain(nt, 0, tail)
        scores[pl.ds(nt * _C, _L)] = dot_group(0, 0, tail)

        pltpu.sync_copy(scores.at[pl.ds(0, epw)], out.at[pl.ds(base, epw)])

    return ker


def kernel(embedding, edge_index):
    E = edge_index.shape[1]
    N, D = embedding.shape
    ei = edge_index.astype(jnp.int32)
    packed = jax.lax.bitcast_convert_type(
        embedding.astype(jnp.bfloat16).reshape(N, D // 2, 2), jnp.int32)
    out = _make_kernel(E, N)(packed, ei[0], ei[1])
    return out[:, None]


# speedup vs baseline: 1.3341x; 1.3341x over previous
import functools

import jax
import jax.numpy as jnp
from jax import lax
from jax.experimental import pallas as pl
from jax.experimental.pallas import tpu as pltpu
from jax.experimental.pallas import tpu_sc as plsc

_NC = 2
_NS = 16
_NW = _NC * _NS
_L = 16
_C = 48
_NB = 3
_D = 256
_DW = _D // 2


@functools.lru_cache(maxsize=None)
def _make_kernel(E, N):
    epw = E // _NW
    nt = epw // _C
    tail = epw - nt * _C
    assert E % _NW == 0 and tail % 8 == 0 and 0 < tail <= _L
    mesh = plsc.VectorSubcoreMesh(core_axis_name="c", subcore_axis_name="s")

    @functools.partial(
        pl.kernel,
        out_type=jax.ShapeDtypeStruct((E,), jnp.float32),
        mesh=mesh,
        compiler_params=pltpu.CompilerParams(
            needs_layout_passes=False,
            internal_scratch_in_bytes=1024 * 1024),
        scratch_types=[
            pltpu.VMEM((epw,), jnp.int32),
            pltpu.VMEM((epw,), jnp.int32),
            pltpu.VMEM((_NB, _C, _D), jnp.float32),
            pltpu.VMEM((_NB, _C, _D), jnp.float32),
            pltpu.VMEM((epw + _L - tail,), jnp.float32),
            pltpu.VMEM((_L * _L,), jnp.float32),
        ] + [pltpu.SemaphoreType.DMA] * _NB,
    )
    def ker(emb, src, dst, out, sidx, didx, srows, drows, scores, accbuf,
            *sems):
        sid = lax.axis_index("s")
        wid = sid * _NC + lax.axis_index("c")
        base = wid * epw

        pltpu.sync_copy(src.at[pl.ds(base, epw)], sidx)
        pltpu.sync_copy(dst.at[pl.ds(base, epw)], didx)

        def fire(g, b, n):
            pltpu.async_copy(emb.at[sidx.at[pl.ds(g * _C, n)]],
                             srows.at[b, pl.ds(0, n)], sems[b])
            pltpu.async_copy(emb.at[didx.at[pl.ds(g * _C, n)]],
                             drows.at[b, pl.ds(0, n)], sems[b])

        def drain(g, b, n):
            pltpu.make_async_copy(emb.at[sidx.at[pl.ds(g * _C, n)]],
                                  srows.at[b, pl.ds(0, n)], sems[b]).wait()
            pltpu.make_async_copy(emb.at[didx.at[pl.ds(g * _C, n)]],
                                  drows.at[b, pl.ds(0, n)], sems[b]).wait()

        def dot_group(b, j, lanes):
            for m in range(lanes):
                e = j * _L + m
                acc = None
                for k in range(_D // _L):
                    prod = (srows[b, e, pl.ds(k * _L, _L)] *
                            drows[b, e, pl.ds(k * _L, _L)])
                    acc = prod if acc is None else acc + prod
                accbuf[pl.ds(m * _L, _L)] = acc
            iot = lax.iota(jnp.int32, _L) * _L
            svec = plsc.load_gather(accbuf, [iot])
            for l in range(1, _L):
                svec = svec + plsc.load_gather(accbuf, [iot + l])
            return svec

        for p in range(_NB - 1):
            if p < nt:
                fire(p, p, _C)

        @pl.loop(0, nt, step=_NB)
        def _chunks(t):
            for b in range(_NB):
                g = t + b

                @pl.when(g < nt)
                def _():
                    @pl.when(g + _NB - 1 < nt)
                    def _():
                        fire(g + _NB - 1, (b + _NB - 1) % _NB, _C)

                    drain(g, b, _C)

                    @pl.loop(0, _C // _L)
                    def _groups(j):
                        scores[pl.ds(g * _C + j * _L, _L)] = \
                            dot_group(b, j, _L)

        fire(nt, 0, tail)
        drain(nt, 0, tail)
        scores[pl.ds(nt * _C, _L)] = dot_group(0, 0, tail)

        pltpu.sync_copy(scores.at[pl.ds(0, epw)], out.at[pl.ds(base, epw)])

    return ker


def kernel(embedding, edge_index):
    E = edge_index.shape[1]
    ei = edge_index.astype(jnp.int32)
    out = _make_kernel(E, embedding.shape[0])(embedding, ei[0], ei[1])
    return out[:, None]


# speedup vs baseline: 1.3637x; 1.0222x over previous
import functools

import jax
import jax.numpy as jnp
from jax import lax
from jax.experimental import pallas as pl
from jax.experimental.pallas import tpu as pltpu
from jax.experimental.pallas import tpu_sc as plsc

_NC = 2
_NS = 16
_NW = _NC * _NS
_L = 16
_C = 48
_NB = 3
_D = 256
_DW = _D // 2


@functools.lru_cache(maxsize=None)
def _make_kernel(E, N):
    epw = E // _NW
    nt = epw // _C
    tail = epw - nt * _C
    assert E % _NW == 0 and tail % 8 == 0 and 0 < tail <= _L
    mesh = plsc.VectorSubcoreMesh(core_axis_name="c", subcore_axis_name="s")

    @functools.partial(
        pl.kernel,
        out_type=jax.ShapeDtypeStruct((E,), jnp.float32),
        mesh=mesh,
        compiler_params=pltpu.CompilerParams(
            needs_layout_passes=False,
            internal_scratch_in_bytes=1024 * 1024),
        scratch_types=[
            pltpu.VMEM((epw,), jnp.int32),
            pltpu.VMEM((epw,), jnp.int32),
            pltpu.VMEM((_NB, _C, _DW), jnp.int32),
            pltpu.VMEM((_NB, _C, _DW), jnp.int32),
            pltpu.VMEM((epw + _L - tail,), jnp.float32),
            pltpu.VMEM((_L * _L,), jnp.float32),
        ] + [pltpu.SemaphoreType.DMA] * _NB,
    )
    def ker(emb, src, dst, out, sidx, didx, srows, drows, scores, accbuf,
            *sems):
        sid = lax.axis_index("s")
        wid = sid * _NC + lax.axis_index("c")
        base = wid * epw

        pltpu.sync_copy(src.at[pl.ds(base, epw)], sidx)
        pltpu.sync_copy(dst.at[pl.ds(base, epw)], didx)

        def fire(g, b, n):
            pltpu.async_copy(emb.at[sidx.at[pl.ds(g * _C, n)]],
                             srows.at[b, pl.ds(0, n)], sems[b])
            pltpu.async_copy(emb.at[didx.at[pl.ds(g * _C, n)]],
                             drows.at[b, pl.ds(0, n)], sems[b])

        def drain(g, b, n):
            pltpu.make_async_copy(emb.at[sidx.at[pl.ds(g * _C, n)]],
                                  srows.at[b, pl.ds(0, n)], sems[b]).wait()
            pltpu.make_async_copy(emb.at[didx.at[pl.ds(g * _C, n)]],
                                  drows.at[b, pl.ds(0, n)], sems[b]).wait()

        def dot_group(b, j, lanes):
            for m in range(lanes):
                e = j * _L + m
                acc = None
                for k in range(_DW // _L):
                    sv = plsc.bitcast(srows[b, e, pl.ds(k * _L, _L)],
                                      jnp.bfloat16)
                    dv = plsc.bitcast(drows[b, e, pl.ds(k * _L, _L)],
                                      jnp.bfloat16)
                    prod = sv * dv
                    acc = prod if acc is None else acc + prod
                u, v = plsc.unpack(acc, format=plsc.PackFormat.INTERLEAVED)
                accbuf[pl.ds(m * _L, _L)] = u + v
            iot = lax.iota(jnp.int32, _L) * _L
            svec = plsc.load_gather(accbuf, [iot])
            for l in range(1, _L):
                svec = svec + plsc.load_gather(accbuf, [iot + l])
            return svec

        for p in range(_NB - 1):
            if p < nt:
                fire(p, p, _C)

        @pl.loop(0, nt, step=_NB)
        def _chunks(t):
            for b in range(_NB):
                g = t + b

                @pl.when(g < nt)
                def _():
                    @pl.when(g + _NB - 1 < nt)
                    def _():
                        fire(g + _NB - 1, (b + _NB - 1) % _NB, _C)

                    drain(g, b, _C)

                    @pl.loop(0, _C // _L)
                    def _groups(j):
                        scores[pl.ds(g * _C + j * _L, _L)] = \
                            dot_group(b, j, _L)

        fire(nt, 0, tail)
        drain(nt, 0, tail)
        scores[pl.ds(nt * _C, _L)] = dot_group(0, 0, tail)

        pltpu.sync_copy(scores.at[pl.ds(0, epw)], out.at[pl.ds(base, epw)])

    return ker


def kernel(embedding, edge_index):
    E = edge_index.shape[1]
    N, D = embedding.shape
    ei = edge_index.astype(jnp.int32)
    packed = jax.lax.bitcast_convert_type(
        embedding.astype(jnp.bfloat16).reshape(N, D // 2, 2), jnp.int32)
    out = _make_kernel(E, N)(packed, ei[0], ei[1])
    return out[:, None]
